# baseline (speedup 1.0000x reference)
import functools
import math

import jax
import jax.numpy as jnp
from jax import lax
from jax.experimental import pallas as pl
from jax.experimental.pallas import tpu as pltpu
from jax.experimental.pallas import tpu_sc as plsc

D_MODEL = 64
NUM_HEADS = 8
SCALE = math.sqrt(D_MODEL)
BATCH, SEQ = 2, 128
B = BATCH * SEQ * SEQ
VOCAB = 34
NC, NS = 2, 16
NW = NC * NS
B_PER_W = B // NW
P_PER_W = B_PER_W // 2
LANES = 16

_OUT4 = (BATCH * NUM_HEADS, SEQ, SEQ, D_MODEL)
SRC_ROWS = B // _OUT4[0]
PAIR_ROWS = SRC_ROWS // 2
SQ_ROWS = 1160


@functools.partial(
    pl.kernel,
    out_type=jax.ShapeDtypeStruct((B // 2, 2 * D_MODEL), jnp.float32),
    mesh=plsc.VectorSubcoreMesh(core_axis_name="c", subcore_axis_name="s"),
    scratch_types=[
        pltpu.VMEM((P_PER_W,), jnp.int32),
        pltpu.VMEM((P_PER_W, 2 * D_MODEL), jnp.float32),
        pltpu.SemaphoreType.DMA,
        pltpu.SemaphoreType.DMA,
    ],
)
def _sc_gather_pairs(sqv, pidx, g_out, pidx_v, buf, gsem, ssem):
    wid = lax.axis_index("s") * NC + lax.axis_index("c")
    pltpu.sync_copy(pidx.at[pl.ds(wid * P_PER_W, P_PER_W)], pidx_v)
    pltpu.async_copy(sqv.at[pidx_v], buf, gsem).wait()
    pltpu.async_copy(buf, g_out.at[pl.ds(wid * P_PER_W, P_PER_W)], ssem).wait()


def _tc_gather_k_body(idx_ref, tab_ref, out_ref):
    idxv = idx_ref[0, 0, :]
    onehot = (
        idxv[:, None] == lax.broadcasted_iota(jnp.int32, (SRC_ROWS, 128), 1)
    ).astype(jnp.float32)
    rows = jnp.dot(
        onehot, tab_ref[...],
        preferred_element_type=jnp.float32,
        precision=lax.Precision.HIGHEST,
    )
    out_ref[...] = jnp.broadcast_to(
        rows[:, None, :], (SRC_ROWS, NUM_HEADS, D_MODEL)
    ).reshape(1, SEQ, SEQ, D_MODEL)


_tc_gather_k = pl.pallas_call(
    _tc_gather_k_body,
    grid=(_OUT4[0],),
    in_specs=[
        pl.BlockSpec((1, 1, SRC_ROWS), lambda n: (n, 0, 0)),
        pl.BlockSpec((128, D_MODEL), lambda n: (0, 0)),
    ],
    out_specs=pl.BlockSpec((1, SEQ, SEQ, D_MODEL), lambda n: (n, 0, 0, 0)),
    out_shape=jax.ShapeDtypeStruct(_OUT4, jnp.float32),
)


def _tc_expand_v_body(g_ref, out_ref):
    pairs = g_ref[...].reshape(PAIR_ROWS, 2, D_MODEL)
    out_ref[...] = jnp.broadcast_to(
        pairs[:, :, None, :], (PAIR_ROWS, 2, NUM_HEADS, D_MODEL)
    ).reshape(1, SEQ, SEQ, D_MODEL)


_tc_expand_v = pl.pallas_call(
    _tc_expand_v_body,
    grid=(_OUT4[0],),
    in_specs=[pl.BlockSpec((PAIR_ROWS, 2 * D_MODEL), lambda n: (n, 0))],
    out_specs=pl.BlockSpec((1, SEQ, SEQ, D_MODEL), lambda n: (n, 0, 0, 0)),
    out_shape=jax.ShapeDtypeStruct(_OUT4, jnp.float32),
)


def kernel(inputs, relation_type, parent_emb, brother_emb):
    if isinstance(relation_type, str) and relation_type == "parent":
        table = parent_emb
    else:
        table = brother_emb
    table = table.at[1].set(0.0) * SCALE
    idx = inputs.reshape(B)
    tabv = table[:, D_MODEL:]
    sqv = jnp.concatenate(
        [
            jnp.broadcast_to(tabv[:, None, :], (VOCAB, VOCAB, D_MODEL)),
            jnp.broadcast_to(tabv[None, :, :], (VOCAB, VOCAB, D_MODEL)),
        ],
        axis=-1,
    ).reshape(VOCAB * VOCAB, 2 * D_MODEL)
    sqv = jnp.pad(sqv, ((0, SQ_ROWS - VOCAB * VOCAB), (0, 0)))
    pidx = idx[0::2] * VOCAB + idx[1::2]
    tabk_pad = jnp.zeros((128, D_MODEL), jnp.float32).at[:VOCAB].set(
        table[:, :D_MODEL]
    )
    k4 = _tc_gather_k(inputs.reshape(_OUT4[0], 1, SRC_ROWS), tabk_pad)
    g2 = _sc_gather_pairs(sqv, pidx)
    v4 = _tc_expand_v(g2)
    return (k4, v4)
